# baseline (speedup 1.0000x reference)
import jax
import jax.numpy as jnp
from jax.experimental import pallas as pl
from jax.experimental.pallas import tpu as pltpu

_CHUNK = 256

_F32 = jnp.float32
_BF16 = jnp.bfloat16


def _fastkv_kernel(x_ref, wq_ref, wk_ref, wv_ref, wo_ref, o_ref,
                   s_ref, wqb_ref, wkb_ref, wvb_ref, wob_ref):
    b = pl.program_id(0)
    c = pl.program_id(1)

    @pl.when(jnp.logical_and(b == 0, c == 0))
    def _():
        wqb_ref[...] = wq_ref[...].astype(_BF16)
        wkb_ref[...] = wk_ref[...].astype(_BF16)
        wvb_ref[...] = wv_ref[...].astype(_BF16)
        wob_ref[...] = wo_ref[...].astype(_BF16)

    @pl.when(c == 0)
    def _():
        s_ref[...] = jnp.zeros_like(s_ref)

    xb = x_ref[0].astype(_BF16)
    q = jax.lax.dot_general(xb, wqb_ref[...], (((1,), (1,)), ((), ())),
                            preferred_element_type=_F32)
    k = jax.lax.dot_general(xb, wkb_ref[...], (((1,), (1,)), ((), ())),
                            preferred_element_type=_F32)
    v = jax.lax.dot_general(xb, wvb_ref[...], (((1,), (1,)), ((), ())),
                            preferred_element_type=_F32)
    qb = q.astype(_BF16)
    kb = k.astype(_BF16)
    vb = v.astype(_BF16)

    y = jnp.dot(qb, s_ref[...].astype(_BF16), preferred_element_type=_F32)

    a = jax.lax.dot_general(qb, kb, (((1,), (1,)), ((), ())),
                            preferred_element_type=_F32)
    i = jax.lax.broadcasted_iota(jnp.int32, a.shape, 0)
    j = jax.lax.broadcasted_iota(jnp.int32, a.shape, 1)
    a = jnp.where(i > j, a, 0.0).astype(_BF16)
    y = y + jnp.dot(a, vb, preferred_element_type=_F32)

    s_ref[...] = s_ref[...] + jax.lax.dot_general(
        kb, vb, (((0,), (0,)), ((), ())), preferred_element_type=_F32)

    o_ref[0] = jax.lax.dot_general(y.astype(_BF16), wob_ref[...],
                                   (((1,), (1,)), ((), ())),
                                   preferred_element_type=_F32)


def kernel(x, Wq, Wk, Wv, Wo):
    B, T, D = x.shape
    DKV = Wq.shape[0]
    C = _CHUNK
    return pl.pallas_call(
        _fastkv_kernel,
        out_shape=jax.ShapeDtypeStruct((B, T, D), x.dtype),
        grid=(B, T // C),
        in_specs=[
            pl.BlockSpec((1, C, D), lambda b, c: (b, c, 0)),
            pl.BlockSpec((DKV, D), lambda b, c: (0, 0)),
            pl.BlockSpec((DKV, D), lambda b, c: (0, 0)),
            pl.BlockSpec((DKV, D), lambda b, c: (0, 0)),
            pl.BlockSpec((D, DKV), lambda b, c: (0, 0)),
        ],
        out_specs=pl.BlockSpec((1, C, D), lambda b, c: (b, c, 0)),
        scratch_shapes=[
            pltpu.VMEM((DKV, DKV), _F32),
            pltpu.VMEM((DKV, D), _BF16),
            pltpu.VMEM((DKV, D), _BF16),
            pltpu.VMEM((DKV, D), _BF16),
            pltpu.VMEM((D, DKV), _BF16),
        ],
        compiler_params=pltpu.CompilerParams(
            dimension_semantics=("parallel", "arbitrary"),
        ),
        name="fastkv_chunked",
    )(x, Wq, Wk, Wv, Wo)


# speedup vs baseline: 1.0296x; 1.0296x over previous
import jax
import jax.numpy as jnp
from jax.experimental import pallas as pl
from jax.experimental.pallas import tpu as pltpu

_CHUNK = 256

_F32 = jnp.float32


def _fastkv_kernel(x_ref, wq_ref, wk_ref, wv_ref, wo_ref, o_ref,
                   s_ref, wqkv_ref, mask_ref):
    c = pl.program_id(0)
    nb, C, D = x_ref.shape
    DKV = wq_ref.shape[0]
    R = nb * C

    @pl.when(c == 0)
    def _():
        s_ref[...] = jnp.zeros_like(s_ref)
        wqkv_ref[0:DKV] = wq_ref[...]
        wqkv_ref[DKV:2 * DKV] = wk_ref[...]
        wqkv_ref[2 * DKV:3 * DKV] = wv_ref[...]
        i = jax.lax.broadcasted_iota(jnp.int32, (R, R), 0)
        j = jax.lax.broadcasted_iota(jnp.int32, (R, R), 1)
        keep = jnp.logical_and(i // C == j // C, i > j)
        mask_ref[...] = jnp.where(keep, 1.0, 0.0)

    xc = x_ref[...].reshape(R, D)
    qkv = jax.lax.dot_general(xc, wqkv_ref[...], (((1,), (1,)), ((), ())),
                              preferred_element_type=_F32)
    q = qkv[:, 0:DKV]
    k = qkv[:, DKV:2 * DKV]
    v = qkv[:, 2 * DKV:3 * DKV]

    a = jax.lax.dot_general(q, k, (((1,), (1,)), ((), ())),
                            preferred_element_type=_F32)
    a = a * mask_ref[...]
    y = jnp.dot(a, v, preferred_element_type=_F32)

    y_inter = []
    for b in range(nb):
        lo, hi = b * C, (b + 1) * C
        y_inter.append(jnp.dot(q[lo:hi], s_ref[b],
                               preferred_element_type=_F32))
        s_ref[b] = s_ref[b] + jax.lax.dot_general(
            k[lo:hi], v[lo:hi], (((0,), (0,)), ((), ())),
            preferred_element_type=_F32)
    y = y + jnp.concatenate(y_inter, axis=0)

    out = jax.lax.dot_general(y, wo_ref[...], (((1,), (1,)), ((), ())),
                              preferred_element_type=_F32)
    o_ref[...] = out.reshape(nb, C, D)


def kernel(x, Wq, Wk, Wv, Wo):
    B, T, D = x.shape
    DKV = Wq.shape[0]
    C = _CHUNK
    R = B * C
    return pl.pallas_call(
        _fastkv_kernel,
        out_shape=jax.ShapeDtypeStruct((B, T, D), x.dtype),
        grid=(T // C,),
        in_specs=[
            pl.BlockSpec((B, C, D), lambda c: (0, c, 0)),
            pl.BlockSpec((DKV, D), lambda c: (0, 0)),
            pl.BlockSpec((DKV, D), lambda c: (0, 0)),
            pl.BlockSpec((DKV, D), lambda c: (0, 0)),
            pl.BlockSpec((D, DKV), lambda c: (0, 0)),
        ],
        out_specs=pl.BlockSpec((B, C, D), lambda c: (0, c, 0)),
        scratch_shapes=[
            pltpu.VMEM((B, DKV, DKV), _F32),
            pltpu.VMEM((3 * DKV, D), _F32),
            pltpu.VMEM((R, R), _F32),
        ],
        compiler_params=pltpu.CompilerParams(
            dimension_semantics=("arbitrary",),
            vmem_limit_bytes=56 * 1024 * 1024,
        ),
        name="fastkv_chunked",
    )(x, Wq, Wk, Wv, Wo)
